# Initial kernel scaffold; baseline (speedup 1.0000x reference)
import jax
import jax.numpy as jnp
from jax.experimental import pallas as pl


def kernel(hidden_states, gate_w, w_gate, w_up, w_down, sw_gate, sw_up, sw_down):
    raise NotImplementedError("write your pallas kernel here")


# speedup vs baseline: 2.1442x; 2.1442x over previous
import jax
import jax.numpy as jnp
from jax.experimental import pallas as pl
from jax.experimental.pallas import tpu as pltpu

E = 8
H = 1024
I_DIM = 512
IS_DIM = 512
TILE_M = 256


def _moe_body(x32_ref, xb_ref, gate_w_ref, wg_ref, wu_ref, wd_ref,
              swg_ref, swu_ref, swd_ref, out_ref):
    x32 = x32_ref[...]
    xb = xb_ref[...]

    logits = jax.lax.dot_general(
        x32, gate_w_ref[...],
        (((1,), (1,)), ((), ())),
        preferred_element_type=jnp.float32)
    m = jnp.max(logits, axis=-1, keepdims=True)
    p = jnp.exp(logits - m)
    p = p / jnp.sum(p, axis=-1, keepdims=True)
    v1 = jnp.max(p, axis=-1, keepdims=True)
    p2 = jnp.where(p >= v1, -jnp.inf, p)
    v2 = jnp.max(p2, axis=-1, keepdims=True)
    s = v1 + v2 + 1e-20
    combine = jnp.where(p >= v1, v1 / s, jnp.where(p >= v2, v2 / s, 0.0))

    acc = jnp.zeros((x32.shape[0], H), jnp.float32)
    for e in range(E):
        g = jnp.dot(xb, wg_ref[e], preferred_element_type=jnp.float32)
        u = jnp.dot(xb, wu_ref[e], preferred_element_type=jnp.float32)
        h = (g * jax.nn.sigmoid(g)) * u * combine[:, e:e + 1]
        acc = acc + jnp.dot(h.astype(jnp.bfloat16), wd_ref[e],
                            preferred_element_type=jnp.float32)

    g = jnp.dot(xb, swg_ref[...], preferred_element_type=jnp.float32)
    u = jnp.dot(xb, swu_ref[...], preferred_element_type=jnp.float32)
    h = (g * jax.nn.sigmoid(g)) * u
    acc = acc + jnp.dot(h.astype(jnp.bfloat16), swd_ref[...],
                        preferred_element_type=jnp.float32)
    out_ref[...] = acc


def kernel(hidden_states, gate_w, w_gate, w_up, w_down, sw_gate, sw_up, sw_down):
    b, s, h = hidden_states.shape
    x = hidden_states.reshape(s, h)
    xb = x.astype(jnp.bfloat16)
    wg = w_gate.astype(jnp.bfloat16)
    wu = w_up.astype(jnp.bfloat16)
    wd = w_down.astype(jnp.bfloat16)
    swg = sw_gate.astype(jnp.bfloat16)
    swu = sw_up.astype(jnp.bfloat16)
    swd = sw_down.astype(jnp.bfloat16)

    out = pl.pallas_call(
        _moe_body,
        grid=(s // TILE_M,),
        in_specs=[
            pl.BlockSpec((TILE_M, H), lambda t: (t, 0)),
            pl.BlockSpec((TILE_M, H), lambda t: (t, 0)),
            pl.BlockSpec((E, H), lambda t: (0, 0)),
            pl.BlockSpec((E, H, I_DIM), lambda t: (0, 0, 0)),
            pl.BlockSpec((E, H, I_DIM), lambda t: (0, 0, 0)),
            pl.BlockSpec((E, I_DIM, H), lambda t: (0, 0, 0)),
            pl.BlockSpec((H, IS_DIM), lambda t: (0, 0)),
            pl.BlockSpec((H, IS_DIM), lambda t: (0, 0)),
            pl.BlockSpec((IS_DIM, H), lambda t: (0, 0)),
        ],
        out_specs=pl.BlockSpec((TILE_M, H), lambda t: (t, 0)),
        out_shape=jax.ShapeDtypeStruct((s, h), jnp.float32),
    )(x, xb, gate_w, wg, wu, wd, swg, swu, swd)
    return out.reshape(b, s, h)


# speedup vs baseline: 2.2006x; 1.0263x over previous
import jax
import jax.numpy as jnp
from jax.experimental import pallas as pl
from jax.experimental.pallas import tpu as pltpu

E = 8
H = 1024
I_DIM = 512
IS_DIM = 512
T = 2048

_HI = jax.lax.Precision.HIGHEST


def _silu_mul(g, u):
    return (g * jax.nn.sigmoid(g)) * u


def _moe_body(x_ref, xb_ref, gate_w_ref, wg_ref, wu_ref, wd_ref,
              swg_ref, swu_ref, swd_ref, out_ref, comb_ref, acc_ref):
    e = pl.program_id(0)

    @pl.when(e == 0)
    def _router():
        x32 = x_ref[...]
        logits = jax.lax.dot_general(
            x32, gate_w_ref[...], (((1,), (1,)), ((), ())),
            preferred_element_type=jnp.float32)
        m = jnp.max(logits, axis=-1, keepdims=True)
        p = jnp.exp(logits - m)
        p = p / jnp.sum(p, axis=-1, keepdims=True)
        v1 = jnp.max(p, axis=-1, keepdims=True)
        p2 = jnp.where(p >= v1, -jnp.inf, p)
        v2 = jnp.max(p2, axis=-1, keepdims=True)
        s = v1 + v2 + 1e-20
        comb_ref[...] = jnp.where(p >= v1, v1 / s,
                                  jnp.where(p >= v2, v2 / s, 0.0))
        acc_ref[...] = jnp.zeros((T, H), jnp.float32)

    xb = xb_ref[...]

    @pl.when(e < E)
    def _routed():
        g = jnp.dot(xb, wg_ref[0], preferred_element_type=jnp.float32)
        u = jnp.dot(xb, wu_ref[0], preferred_element_type=jnp.float32)
        lane = jax.lax.broadcasted_iota(jnp.int32, (T, E), 1)
        col = jnp.sum(jnp.where(lane == e, comb_ref[...], 0.0),
                      axis=-1, keepdims=True)
        h = _silu_mul(g, u) * col
        acc_ref[...] += jnp.dot(h.astype(jnp.bfloat16), wd_ref[0],
                                preferred_element_type=jnp.float32)

    @pl.when(e == E)
    def _shared():
        g = jnp.dot(xb, swg_ref[...], preferred_element_type=jnp.float32)
        u = jnp.dot(xb, swu_ref[...], preferred_element_type=jnp.float32)
        h = _silu_mul(g, u)
        out_ref[...] = acc_ref[...] + jnp.dot(
            h.astype(jnp.bfloat16), swd_ref[...],
            preferred_element_type=jnp.float32)


def kernel(hidden_states, gate_w, w_gate, w_up, w_down, sw_gate, sw_up, sw_down):
    b, s, h = hidden_states.shape
    x = hidden_states.reshape(s, h)
    xb = x.astype(jnp.bfloat16)
    wg = w_gate.astype(jnp.bfloat16)
    wu = w_up.astype(jnp.bfloat16)
    wd = w_down.astype(jnp.bfloat16)
    swg = sw_gate.astype(jnp.bfloat16)
    swu = sw_up.astype(jnp.bfloat16)
    swd = sw_down.astype(jnp.bfloat16)

    out = pl.pallas_call(
        _moe_body,
        grid=(E + 1,),
        in_specs=[
            pl.BlockSpec((T, H), lambda e: (0, 0)),
            pl.BlockSpec((T, H), lambda e: (0, 0)),
            pl.BlockSpec((E, H), lambda e: (0, 0)),
            pl.BlockSpec((1, H, I_DIM), lambda e: (jnp.minimum(e, E - 1), 0, 0)),
            pl.BlockSpec((1, H, I_DIM), lambda e: (jnp.minimum(e, E - 1), 0, 0)),
            pl.BlockSpec((1, I_DIM, H), lambda e: (jnp.minimum(e, E - 1), 0, 0)),
            pl.BlockSpec((H, IS_DIM), lambda e: (0, 0)),
            pl.BlockSpec((H, IS_DIM), lambda e: (0, 0)),
            pl.BlockSpec((IS_DIM, H), lambda e: (0, 0)),
        ],
        out_specs=pl.BlockSpec((T, H), lambda e: (0, 0)),
        out_shape=jax.ShapeDtypeStruct((s, h), jnp.float32),
        scratch_shapes=[
            pltpu.VMEM((T, E), jnp.float32),
            pltpu.VMEM((T, H), jnp.float32),
        ],
    )(x, xb, gate_w, wg, wu, wd, swg, swu, swd)
    return out.reshape(b, s, h)
